# baseline (speedup 1.0000x reference)
import functools

import jax
import jax.numpy as jnp
from jax import lax
from jax.experimental import pallas as pl
from jax.experimental.pallas import tpu as pltpu
from jax.experimental.pallas import tpu_sc as plsc

N = 100000
D = 128
B = 256
C = 10
R = 2000
NSUB = 5
RB = R * NSUB
K = N // RB
W = 32
BP = B + W
SP = 384

CH = 80
NCHUNKS = N // CH
NW = 32
NBUF = 4

F32 = jnp.float32

NFULL = NCHUNKS // NW
NEXTRA = NCHUNKS % NW


def _sc_pooled_body(x_hbm, ids_hbm, zeros_hbm, out_hbm, *refs):
    xvs = [refs[2 * b] for b in range(NBUF)]
    ivs = [refs[2 * b + 1] for b in range(NBUF)]
    acc_sh = refs[2 * NBUF]
    base_s = 2 * NBUF + 1
    semx = refs[base_s: base_s + NBUF]
    semi = refs[base_s + NBUF: base_s + 2 * NBUF]
    sems = refs[base_s + 2 * NBUF: base_s + 3 * NBUF]

    cid = lax.axis_index("c")
    sid = lax.axis_index("s")
    wid = sid * 2 + cid

    @pl.when(sid == 0)
    def _():
        pltpu.sync_copy(zeros_hbm, acc_sh)

    plsc.subcore_barrier()

    def start_in(j, b):
        r0 = (wid + j * NW) * CH
        pltpu.async_copy(x_hbm.at[pl.ds(r0, CH)], xvs[b], semx[b])
        pltpu.async_copy(ids_hbm.at[pl.ds(r0, CH)], ivs[b], semi[b])

    def wait_in(b):
        pltpu.make_async_copy(x_hbm.at[pl.ds(0, CH)], xvs[b], semx[b]).wait()
        pltpu.make_async_copy(ids_hbm.at[pl.ds(0, CH)], ivs[b], semi[b]).wait()

    def scatter_start(b):
        pltpu.async_copy(xvs[b], acc_sh.at[ivs[b]], sems[b], add=True)

    def scatter_wait(b):
        pltpu.make_async_copy(xvs[b], acc_sh.at[ivs[b]], sems[b]).wait()

    for b in range(NBUF - 1):
        start_in(b, b)

    def body(i, carry):
        for jj in range(NBUF):
            j = NBUF * i + jj
            wait_in(jj)
            scatter_start(jj)

            @pl.when(j >= 1)
            def _():
                scatter_wait((jj - 1) % NBUF)

            start_in(j + NBUF - 1, (jj + NBUF - 1) % NBUF)
        return carry

    nloops = (NFULL - (NBUF - 1)) // NBUF
    lax.fori_loop(0, nloops, body, 0)
    for jj in range(NBUF * nloops, NFULL):
        b = jj % NBUF
        wait_in(b)
        scatter_start(b)
        scatter_wait((b - 1) % NBUF)
    scatter_wait((NFULL - 1) % NBUF)

    @pl.when(wid < NEXTRA)
    def _():
        r0 = (NW * NFULL + wid) * CH
        pltpu.sync_copy(x_hbm.at[pl.ds(r0, CH)], xvs[0])
        pltpu.sync_copy(ids_hbm.at[pl.ds(r0, CH)], ivs[0])
        pltpu.sync_copy(xvs[0], acc_sh.at[ivs[0]], add=True)

    plsc.subcore_barrier()

    @pl.when(sid == 0)
    def _():
        pltpu.sync_copy(acc_sh, out_hbm.at[cid])


assert NFULL % NBUF == NBUF - 1


@functools.cache
def _get_sc_pooled():
    return functools.partial(
        pl.kernel,
        out_type=jax.ShapeDtypeStruct((2, BP, D), F32),
        mesh=plsc.VectorSubcoreMesh(core_axis_name="c",
                                    subcore_axis_name="s"),
        scratch_types=(
            [pltpu.VMEM((CH, D), F32), pltpu.VMEM((CH,), jnp.int32)] * NBUF
            + [pltpu.VMEM_SHARED((BP, D), F32)]
            + [pltpu.SemaphoreType.DMA] * (3 * NBUF)
        ),
    )(_sc_pooled_body)


def _tc_body(lo_ref, hi_ref, x_ref, scol_ref, pooled2_ref,
             Wh0_ref, bh0_ref, Wh1_ref, bh1_ref, Wh2_ref, bh2_ref,
             Wt0_ref, bt0_ref, Wt1_ref, bt1_ref,
             out_ref, p2_scr, o_scr):
    k = pl.program_id(0)

    @pl.when(k == 0)
    def _():
        pooled = pooled2_ref[0] + pooled2_ref[1]
        p2_scr[...] = jnp.dot(pooled, Wt0_ref[D:, :],
                              preferred_element_type=F32) + bt0_ref[...]
        o_scr[...] = jnp.zeros_like(o_scr)

    for h in range(NSUB):
        sb = k * NSUB + h
        row0 = k * RB + h * R
        xb = x_ref[pl.ds(h * R, R), :]
        h0 = jnp.maximum(jnp.dot(xb, Wh0_ref[...],
                                 preferred_element_type=F32)
                         + bh0_ref[...], 0.0)
        h1 = jnp.maximum(jnp.dot(h0, Wh1_ref[...],
                                 preferred_element_type=F32)
                         + bh1_ref[...], 0.0)
        hv = jnp.dot(h1, Wh2_ref[...], preferred_element_type=F32) \
            + bh2_ref[...]
        hrow = jnp.swapaxes(hv, 0, 1)

        u = jnp.dot(xb, Wt0_ref[:D, :], preferred_element_type=F32)

        lo = lo_ref[sb]
        hi = hi_ref[sb]
        nch = (hi - lo) // W + 1
        riota = row0 + lax.broadcasted_iota(jnp.int32, (R, 1), 0)
        ciota = row0 + lax.broadcasted_iota(jnp.int32, (1, R), 1)

        def g_chunk(c, g, riota=riota, lo=lo):
            base = lo + c * W
            srow = jnp.swapaxes(scol_ref[pl.ds(base, W), :], 0, 1)
            erow = jnp.swapaxes(scol_ref[pl.ds(base + 1, W), :], 0, 1)
            oh = ((riota >= srow) & (riota < erow)).astype(F32)
            return g + jnp.dot(oh, p2_scr[pl.ds(base, W), :],
                               preferred_element_type=F32)

        g = lax.fori_loop(0, nch, g_chunk, u)
        t = jnp.maximum(g, 0.0)
        theta = jnp.dot(t, Wt1_ref[...], preferred_element_type=F32) \
            + bt1_ref[...]

        def s_chunk(c, carry, ciota=ciota, lo=lo, theta=theta, hrow=hrow):
            base = lo + c * W
            scol = scol_ref[pl.ds(base, W), :]
            ecol = scol_ref[pl.ds(base + 1, W), :]
            ohT = jnp.where((ciota >= scol) & (ciota < ecol), hrow, 0.0)
            o_scr[pl.ds(base, W), :] += jnp.dot(ohT, theta,
                                                preferred_element_type=F32)
            return carry

        lax.fori_loop(0, nch, s_chunk, 0)

    @pl.when(k == K - 1)
    def _():
        out_ref[...] = o_scr[0:B, :]


@jax.jit
def kernel(x, batch, Wh0, bh0, Wh1, bh1, Wh2, bh2, Wt0, bt0, Wt1, bt1):
    ids = batch.astype(jnp.int32)
    lo = ids[::R]
    hi = ids[R - 1::R]

    starts = jnp.searchsorted(ids, jnp.arange(B + 1, dtype=jnp.int32),
                              side="left").astype(jnp.int32)
    scol = jnp.concatenate(
        [starts, jnp.full((SP - (B + 1),), N, jnp.int32)]).reshape(SP, 1)

    pooled2 = _get_sc_pooled()(x, ids, jnp.zeros((BP, D), F32))

    row_spec = pl.BlockSpec((RB, D), lambda i, lo, hi: (i, 0))
    full = lambda a: pl.BlockSpec(a.shape, lambda i, lo, hi: (0,) * a.ndim)

    grid = pltpu.PrefetchScalarGridSpec(
        num_scalar_prefetch=2,
        grid=(K,),
        in_specs=[row_spec, full(scol), full(pooled2),
                  full(Wh0), full(bh0), full(Wh1), full(bh1), full(Wh2),
                  full(bh2), full(Wt0), full(bt0), full(Wt1), full(bt1)],
        out_specs=[pl.BlockSpec((B, C), lambda i, lo, hi: (0, 0))],
        scratch_shapes=[pltpu.VMEM((BP, D), F32),
                        pltpu.VMEM((BP, C), F32)],
    )
    out = pl.pallas_call(
        _tc_body,
        grid_spec=grid,
        out_shape=[jax.ShapeDtypeStruct((B, C), F32)],
    )(lo, hi, x, scol, pooled2,
      Wh0, bh0, Wh1, bh1, Wh2, bh2, Wt0, bt0, Wt1, bt1)[0]
    return out
